# baseline (speedup 1.0000x reference)
import functools

import jax
import jax.numpy as jnp
from jax import lax
from jax.experimental import pallas as pl
from jax.experimental.pallas import tpu as pltpu
from jax.experimental.pallas import tpu_sc as plsc

_D = 768
_GRID = 32
_LANES = 16
_NCHUNK = _D // _LANES


def _dembed_body(xT_hbm, tok_hbm, col_hbm, row_hbm, ext_hbm, out_hbm,
                 idx_v, oidx_v, pos_v, tmp_v, rows_v, gsem, ssem):
    N, B = xT_hbm.shape
    nw = 32
    wid = lax.axis_index("s") * 2 + lax.axis_index("c")

    base_trips = N // nw
    trips = jnp.where(wid < (N % nw), base_trips + 1, base_trips)

    def body(t, carry):
        n = wid + nw * t

        pltpu.sync_copy(xT_hbm.at[n], idx_v)

        @pl.when(n == 0)
        def _():
            pltpu.sync_copy(ext_hbm.at[0], pos_v)

        @pl.when(n > 0)
        def _():
            m = n - 1
            pltpu.sync_copy(col_hbm.at[m // _GRID], pos_v)
            pltpu.sync_copy(row_hbm.at[m % _GRID], tmp_v)
            for j in range(_NCHUNK):
                ds = pl.ds(_LANES * j, _LANES)
                pos_v[ds] = pos_v[ds] + tmp_v[ds]

        pltpu.async_copy(tok_hbm.at[idx_v], rows_v, gsem).wait()

        for k in range(B // _LANES):
            oidx_v[pl.ds(_LANES * k, _LANES)] = (
                lax.iota(jnp.int32, _LANES) + (_LANES * k)) * N + n

        def addrow(b, c):
            for j in range(_NCHUNK):
                ds = pl.ds(_LANES * j, _LANES)
                rows_v[b, ds] = rows_v[b, ds] + pos_v[ds]
            return c

        lax.fori_loop(0, B, addrow, 0)

        pltpu.async_copy(rows_v, out_hbm.at[oidx_v], ssem).wait()
        return carry

    lax.fori_loop(0, trips, body, 0)


def kernel(x, tok_table, col_table, row_table, ext_table):
    B, N = x.shape
    xT = x.T

    mesh = plsc.VectorSubcoreMesh(core_axis_name="c", subcore_axis_name="s")
    run = functools.partial(
        pl.kernel,
        out_type=jax.ShapeDtypeStruct((B * N, _D), jnp.float32),
        mesh=mesh,
        scratch_types=[
            pltpu.VMEM((B,), jnp.int32),
            pltpu.VMEM((B,), jnp.int32),
            pltpu.VMEM((_D,), jnp.float32),
            pltpu.VMEM((_D,), jnp.float32),
            pltpu.VMEM((B, _D), jnp.float32),
            pltpu.SemaphoreType.DMA,
            pltpu.SemaphoreType.DMA,
        ],
    )(_dembed_body)
    out = run(xT, tok_table, col_table, row_table, ext_table)
    return out.reshape(B, N, _D)


# speedup vs baseline: 1.9454x; 1.9454x over previous
import functools

import jax
import jax.numpy as jnp
from jax import lax
from jax.experimental import pallas as pl
from jax.experimental.pallas import tpu as pltpu
from jax.experimental.pallas import tpu_sc as plsc

_D = 768
_GRID = 32
_LANES = 16
_NCHUNK = _D // _LANES
_NW = 32
_GROUP = 8


def _dembed_body(xT_hbm, tok_hbm, col_hbm, row_hbm, ext_hbm, out_hbm,
                 idx_v, oidx_v, cbuf_v, rbuf_v, rows_v, gsem, ssem, csem, rsem):
    N, B = xT_hbm.shape
    wid = lax.axis_index("s") * 2 + lax.axis_index("c")
    trips = jnp.where(wid < (N % _NW), N // _NW + 1, N // _NW)

    def gather_desc(s):
        return pltpu.make_async_copy(tok_hbm.at[idx_v.at[s]], rows_v.at[s],
                                     gsem.at[s])

    def scatter_desc(s):
        return pltpu.make_async_copy(rows_v.at[s], out_hbm.at[oidx_v.at[s]],
                                     ssem.at[s])

    def issue_pos_loads(n, s):
        m = n - 1
        pltpu.async_copy(col_hbm.at[m // _GRID], cbuf_v.at[s], csem.at[s])
        pltpu.async_copy(row_hbm.at[m % _GRID], rbuf_v.at[s], rsem.at[s])

    def wait_pos_loads(s):
        pltpu.make_async_copy(col_hbm.at[0], cbuf_v.at[s], csem.at[s]).wait()
        pltpu.make_async_copy(row_hbm.at[0], rbuf_v.at[s], rsem.at[s]).wait()

    def add_pos(s):
        for g in range(_NCHUNK // _GROUP):
            base = g * _GROUP * _LANES
            cvs = tuple(cbuf_v[s, pl.ds(base + _LANES * j, _LANES)]
                        for j in range(_GROUP))
            rvs = tuple(rbuf_v[s, pl.ds(base + _LANES * j, _LANES)]
                        for j in range(_GROUP))

            def rowbody(b, carry, base=base):
                cc, rr = carry
                for j in range(_GROUP):
                    ds = pl.ds(base + _LANES * j, _LANES)
                    rows_v[s, b, ds] = rows_v[s, b, ds] + cc[j] + rr[j]
                return carry

            lax.fori_loop(0, B, rowbody, (cvs, rvs))

    n0 = wid
    pltpu.sync_copy(xT_hbm.at[n0], idx_v.at[0])
    gather_desc(0).start()

    @pl.when(wid == 0)
    def _():
        pltpu.sync_copy(ext_hbm.at[0], cbuf_v.at[0])
        zero = jnp.zeros((_LANES,), jnp.float32)
        for j in range(_NCHUNK):
            rbuf_v[0, pl.ds(_LANES * j, _LANES)] = zero

    @pl.when(wid > 0)
    def _():
        pltpu.sync_copy(col_hbm.at[(wid - 1) // _GRID], cbuf_v.at[0])
        pltpu.sync_copy(row_hbm.at[(wid - 1) % _GRID], rbuf_v.at[0])

    def body(t, carry):
        p = lax.rem(t, 2)
        q = 1 - p
        n = wid + _NW * t

        gather_desc(p).wait()

        @pl.when(t + 1 < trips)
        def _():
            @pl.when(t >= 1)
            def _():
                scatter_desc(q).wait()

            pltpu.sync_copy(xT_hbm.at[n + _NW], idx_v.at[q])
            gather_desc(q).start()
            issue_pos_loads(n + _NW, q)

        @pl.when(t >= 1)
        def _():
            wait_pos_loads(p)

        for k in range(B // _LANES):
            oidx_v[p, pl.ds(_LANES * k, _LANES)] = (
                lax.iota(jnp.int32, _LANES) + (_LANES * k)) * N + n

        add_pos(p)
        scatter_desc(p).start()
        return carry

    lax.fori_loop(0, trips, body, 0)

    scatter_desc(lax.rem(trips - 2, 2)).wait()
    scatter_desc(lax.rem(trips - 1, 2)).wait()


def kernel(x, tok_table, col_table, row_table, ext_table):
    B, N = x.shape
    xT = x.T

    mesh = plsc.VectorSubcoreMesh(core_axis_name="c", subcore_axis_name="s")
    run = functools.partial(
        pl.kernel,
        out_type=jax.ShapeDtypeStruct((B * N, _D), jnp.float32),
        mesh=mesh,
        scratch_types=[
            pltpu.VMEM((2, B), jnp.int32),
            pltpu.VMEM((2, B), jnp.int32),
            pltpu.VMEM((2, _D), jnp.float32),
            pltpu.VMEM((2, _D), jnp.float32),
            pltpu.VMEM((2, B, _D), jnp.float32),
            pltpu.SemaphoreType.DMA((2,)),
            pltpu.SemaphoreType.DMA((2,)),
            pltpu.SemaphoreType.DMA((2,)),
            pltpu.SemaphoreType.DMA((2,)),
        ],
    )(_dembed_body)
    out = run(xT, tok_table, col_table, row_table, ext_table)
    return out.reshape(B, N, _D)


# speedup vs baseline: 2.0198x; 1.0383x over previous
import functools

import jax
import jax.numpy as jnp
from jax import lax
from jax.experimental import pallas as pl
from jax.experimental.pallas import tpu as pltpu
from jax.experimental.pallas import tpu_sc as plsc

_D = 768
_GRID = 32
_LANES = 16
_NCHUNK = _D // _LANES
_NW = 32
_GROUP = 16
_TMAX = 33


def _dembed_body(xP_hbm, tok_hbm, col_hbm, row_hbm, ext_hbm, out_hbm,
                 idxall_v, oidx_v, coltab_v, rowrow_v, pos_v, rows_v,
                 gsem, ssem):
    NWK, TMAX, B = xP_hbm.shape
    V, D = tok_hbm.shape
    N = NWK * (TMAX - 1) + 1
    wid = lax.axis_index("s") * 2 + lax.axis_index("c")
    trips = jnp.where(wid == 0, TMAX, TMAX - 1)

    def gather_desc(t, s):
        return pltpu.make_async_copy(tok_hbm.at[idxall_v.at[t]],
                                     rows_v.at[s], gsem.at[s])

    def scatter_desc(s):
        return pltpu.make_async_copy(rows_v.at[s], out_hbm.at[oidx_v.at[s]],
                                     ssem.at[s])

    pltpu.sync_copy(xP_hbm.at[wid], idxall_v)
    gather_desc(0, 0).start()
    pltpu.sync_copy(col_hbm, coltab_v)
    pltpu.sync_copy(row_hbm.at[lax.rem(wid + _GRID - 1, _GRID)], rowrow_v)

    def body(t, carry):
        p = lax.rem(t, 2)
        q = 1 - p
        n = wid + _NW * t

        gather_desc(t, p).wait()

        @pl.when(t + 1 < trips)
        def _():
            @pl.when(t >= 1)
            def _():
                scatter_desc(q).wait()

            gather_desc(t + 1, q).start()

        @pl.when(n == 0)
        def _():
            pltpu.sync_copy(ext_hbm.at[0], pos_v)

        @pl.when(n > 0)
        def _():
            cidx = jnp.where(wid == 0, t - 1, t)
            for j in range(_NCHUNK):
                ds = pl.ds(_LANES * j, _LANES)
                pos_v[ds] = coltab_v[cidx, ds] + rowrow_v[ds]

        for k in range(B // _LANES):
            oidx_v[p, pl.ds(_LANES * k, _LANES)] = (
                lax.iota(jnp.int32, _LANES) + (_LANES * k)) * N + n

        for g in range(_NCHUNK // _GROUP):
            base = g * _GROUP * _LANES
            pvs = tuple(pos_v[pl.ds(base + _LANES * j, _LANES)]
                        for j in range(_GROUP))

            @plsc.parallel_loop(0, B, carry=pvs)
            def rowbody(b, pv, base=base):
                for j in range(_GROUP):
                    ds = pl.ds(base + _LANES * j, _LANES)
                    rows_v[p, b, ds] = rows_v[p, b, ds] + pv[j]
                return pv

        scatter_desc(p).start()
        return carry

    lax.fori_loop(0, trips, body, 0)

    scatter_desc(lax.rem(trips - 2, 2)).wait()
    scatter_desc(lax.rem(trips - 1, 2)).wait()


def kernel(x, tok_table, col_table, row_table, ext_table):
    B, N = x.shape
    xT = x.T
    xP = jnp.concatenate(
        [xT[: _NW * (_TMAX - 1)].reshape(_TMAX - 1, _NW, B).transpose(1, 0, 2),
         jnp.broadcast_to(xT[_NW * (_TMAX - 1):], (_NW, 1, B))], axis=1)

    mesh = plsc.VectorSubcoreMesh(core_axis_name="c", subcore_axis_name="s")
    run = functools.partial(
        pl.kernel,
        out_type=jax.ShapeDtypeStruct((B * N, _D), jnp.float32),
        mesh=mesh,
        scratch_types=[
            pltpu.VMEM((_TMAX, B), jnp.int32),
            pltpu.VMEM((2, B), jnp.int32),
            pltpu.VMEM((_GRID, _D), jnp.float32),
            pltpu.VMEM((_D,), jnp.float32),
            pltpu.VMEM((_D,), jnp.float32),
            pltpu.VMEM((2, B, _D), jnp.float32),
            pltpu.SemaphoreType.DMA((2,)),
            pltpu.SemaphoreType.DMA((2,)),
        ],
    )(_dembed_body)
    out = run(xP, tok_table, col_table, row_table, ext_table)
    return out.reshape(B, N, _D)
